# baseline (speedup 1.0000x reference)
import functools

import jax
import jax.numpy as jnp
import numpy as np
from jax import lax
from jax.experimental import pallas as pl
from jax.experimental.pallas import tpu as pltpu
from jax.experimental.pallas import tpu_sc as plsc

ANGSTROM_TO_BOHR = 1.8897261258369282
N_NODES = 100000
N_EDGES = 3200000
NUM_ELEMENTS = 100
EMBED_DIM = 16
NUM_RBF = 16
R_MAX_BOHR = 7.56

NC = 2
NS = 16
NW = NC * NS

N_PAD = 102400
ROWS_W = N_PAD // NW
NGROUP_W = ROWS_W // 128
E_PAD = 3211264
EDGES_W = E_PAD // NW
EBLK = 16384
CHUNK = 512
GPC = CHUNK // 128
NCHUNK = EDGES_W // CHUNK

ROWS_TILE = N_PAD // NS
SPAN_PIECES = [(i * CHUNK, CHUNK) for i in range(12)] + [(6144, 256)]


def _mesh():
    return plsc.VectorSubcoreMesh(core_axis_name="c", subcore_axis_name="s",
                                  num_cores=NC, num_subcores=NS)


_SC_PARAMS = pltpu.CompilerParams(use_tc_tiling_on_sc=False)


def _h_gather_body(elem_hbm, table_hbm, h_hbm, idx_a, idx_b, rows_v, sem):
    c = lax.axis_index("c")
    s = lax.axis_index("s")
    wid = c * NS + s
    base = pl.multiple_of(wid * ROWS_W, 128)
    idx = [idx_a, idx_b]
    pltpu.sync_copy(elem_hbm.at[pl.ds(base, 128)], idx_a)
    descs = [None, None]
    for g in range(NGROUP_W):
        nb = (g + 1) % 2
        if g + 1 < NGROUP_W:
            if descs[nb] is not None:
                descs[nb].wait()
                descs[nb] = None
            pltpu.sync_copy(elem_hbm.at[pl.ds(base + (g + 1) * 128, 128)],
                            idx[nb])
        descs[g % 2] = pltpu.async_copy(
            table_hbm.at[idx[g % 2]],
            rows_v.at[pl.ds(g * 128, 128), :], sem)
    for d in descs:
        if d is not None:
            d.wait()
    pltpu.sync_copy(rows_v, h_hbm.at[pl.ds(base, ROWS_W), :])


def _h_gather(elem_pad, table):
    return pl.kernel(
        _h_gather_body,
        out_type=jax.ShapeDtypeStruct((N_PAD, EMBED_DIM), jnp.float32),
        mesh=_mesh(),
        scratch_types=[
            pltpu.VMEM((128,), jnp.int32),
            pltpu.VMEM((128,), jnp.int32),
            pltpu.VMEM((ROWS_W, EMBED_DIM), jnp.float32),
            pltpu.SemaphoreType.DMA,
        ],
        compiler_params=_SC_PARAMS,
    )(elem_pad, table)


_R_MIN = 0.5 * ANGSTROM_TO_BOHR
_WIDTH = (R_MAX_BOHR - _R_MIN) / NUM_RBF
_CENTERS = np.linspace(_R_MIN, R_MAX_BOHR, NUM_RBF, dtype=np.float32)


def _filter_body(d_ref, w1_ref, b1_ref, w2_ref, b2_ref, out_ref):
    d = d_ref[0, 0, :]
    step = np.float32((R_MAX_BOHR - _R_MIN) / (NUM_RBF - 1))
    k = lax.broadcasted_iota(jnp.int32, (EBLK, NUM_RBF), 1).astype(jnp.float32)
    centers = np.float32(_R_MIN) + k * step
    t = (d[:, None] - centers) * np.float32(1.0 / _WIDTH)
    rbf = jnp.exp(np.float32(-0.5) * t * t)
    z = jnp.dot(rbf, w1_ref[...],
                preferred_element_type=jnp.float32) + b1_ref[0, :]
    zs = z * (np.float32(1.0) / (np.float32(1.0) + jnp.exp(-z)))
    out_ref[...] = jnp.dot(zs, w2_ref[...],
                           preferred_element_type=jnp.float32) + b2_ref[0, :]


def _filter_net(d2, W1, b1, W2, b2):
    nblk = E_PAD // EBLK
    return pl.pallas_call(
        _filter_body,
        grid=(nblk,),
        in_specs=[
            pl.BlockSpec((1, 1, EBLK), lambda i: (i, 0, 0)),
            pl.BlockSpec((NUM_RBF, EMBED_DIM), lambda i: (0, 0)),
            pl.BlockSpec((1, EMBED_DIM), lambda i: (0, 0)),
            pl.BlockSpec((EMBED_DIM, EMBED_DIM), lambda i: (0, 0)),
            pl.BlockSpec((1, EMBED_DIM), lambda i: (0, 0)),
        ],
        out_specs=pl.BlockSpec((EBLK, EMBED_DIM), lambda i: (i, 0)),
        out_shape=jax.ShapeDtypeStruct((E_PAD, EMBED_DIM), jnp.float32),
    )(d2, W1, b1.reshape(1, EMBED_DIM), W2, b2.reshape(1, EMBED_DIM))


def _edge_body(src_hbm, tgt_hbm, wf_hbm, h_hbm, p_hbm,
               idx_t, idx_s, rows, wfv, acc, sem):
    c = lax.axis_index("c")
    s = lax.axis_index("s")
    wid = c * NS + s

    def zero_row(i, carry):
        rows[i, :] = jnp.zeros((EMBED_DIM,), jnp.float32)
        return carry
    lax.fori_loop(0, CHUNK, zero_row, 0)
    for off, size in SPAN_PIECES:
        pltpu.sync_copy(rows.at[pl.ds(0, size), :],
                        acc.at[pl.ds(s * ROWS_TILE + off, size), :])
    plsc.subcore_barrier()

    base_group = wid * (EDGES_W // 128)

    @pl.loop(0, NCHUNK)
    def chunk_body(ch):
        grow = base_group + ch * GPC
        ebase = grow * 128
        pltpu.sync_copy(tgt_hbm.at[pl.ds(grow, GPC), :], idx_t)
        pltpu.sync_copy(src_hbm.at[pl.ds(grow, GPC), :], idx_s)
        pltpu.sync_copy(wf_hbm.at[pl.ds(ebase, CHUNK), :], wfv)
        descs = []
        for g in range(GPC):
            descs.append(pltpu.async_copy(
                h_hbm.at[idx_t.at[g]],
                rows.at[pl.ds(g * 128, 128), :], sem))
        for d in descs:
            d.wait()

        @pl.loop(0, CHUNK, unroll=8)
        def mul_row(e):
            rows[e, :] = rows[e, :] * wfv[e, :]

        for g in range(GPC):
            pltpu.sync_copy(rows.at[pl.ds(g * 128, 128), :],
                            acc.at[idx_s.at[g]], add=True)

    plsc.subcore_barrier()

    for off, size in SPAN_PIECES:
        pltpu.sync_copy(acc.at[pl.ds(s * ROWS_TILE + off, size), :],
                        rows.at[pl.ds(0, size), :])
        pltpu.sync_copy(rows.at[pl.ds(0, size), :],
                        p_hbm.at[c, pl.ds(s * ROWS_TILE + off, size), :])


def _edge_aggregate(src2d, tgt2d, wf, h_pad):
    return pl.kernel(
        _edge_body,
        out_type=jax.ShapeDtypeStruct((NC, N_PAD, EMBED_DIM), jnp.float32),
        mesh=_mesh(),
        scratch_types=[
            pltpu.VMEM((GPC, 128), jnp.int32),
            pltpu.VMEM((GPC, 128), jnp.int32),
            pltpu.VMEM((CHUNK, EMBED_DIM), jnp.float32),
            pltpu.VMEM((CHUNK, EMBED_DIM), jnp.float32),
            pltpu.VMEM_SHARED((N_PAD, EMBED_DIM), jnp.float32),
            pltpu.SemaphoreType.DMA,
        ],
        compiler_params=_SC_PARAMS,
    )(src2d, tgt2d, wf, h_pad)


def _combine_body(h_ref, p_ref, out_ref):
    out_ref[...] = h_ref[...] + p_ref[0] + p_ref[1]


def _combine(h_pad, partials):
    h2 = h_pad.reshape(N_PAD // 8, 128)
    p2 = partials.reshape(NC, N_PAD // 8, 128)
    nrow = N_PAD // 8
    blk = 1280
    out = pl.pallas_call(
        _combine_body,
        grid=(nrow // blk,),
        in_specs=[
            pl.BlockSpec((blk, 128), lambda i: (i, 0)),
            pl.BlockSpec((NC, blk, 128), lambda i: (0, i, 0)),
        ],
        out_specs=pl.BlockSpec((blk, 128), lambda i: (i, 0)),
        out_shape=jax.ShapeDtypeStruct((nrow, 128), jnp.float32),
    )(h2, p2)
    return out.reshape(N_PAD, EMBED_DIM)[:N_NODES]


@jax.jit
def kernel(elem_idx, edge_index, distances, embed_table, W1, b1, W2, b2):
    elem_pad = jnp.pad(elem_idx, (0, N_PAD - N_NODES))

    src = edge_index[0]
    tgt = edge_index[1]
    epad = E_PAD - N_EDGES
    src_pad = jnp.pad(src, (0, epad), constant_values=N_NODES)
    tgt_pad = jnp.pad(tgt, (0, epad))
    d_pad = jnp.pad(distances, (0, epad))
    src2d = src_pad.reshape(E_PAD // 128, 128)
    tgt2d = tgt_pad.reshape(E_PAD // 128, 128)
    d2 = d_pad.reshape(E_PAD // EBLK, 1, EBLK)

    h_pad = _h_gather(elem_pad, embed_table)
    wf = _filter_net(d2, W1, b1, W2, b2)
    partials = _edge_aggregate(src2d, tgt2d, wf, h_pad)
    return _combine(h_pad, partials)


# speedup vs baseline: 1.0398x; 1.0398x over previous
import functools

import jax
import jax.numpy as jnp
import numpy as np
from jax import lax
from jax.experimental import pallas as pl
from jax.experimental.pallas import tpu as pltpu
from jax.experimental.pallas import tpu_sc as plsc

ANGSTROM_TO_BOHR = 1.8897261258369282
N_NODES = 100000
N_EDGES = 3200000
NUM_ELEMENTS = 100
EMBED_DIM = 16
NUM_RBF = 16
R_MAX_BOHR = 7.56

NC = 2
NS = 16
NW = NC * NS

N_PAD = 102400
ROWS_W = N_PAD // NW
NGROUP_W = ROWS_W // 128
E_PAD = 3211264
EDGES_W = E_PAD // NW
EBLK = 16384
CHUNK = 512
GPC = CHUNK // 128
NCHUNK = EDGES_W // CHUNK

ROWS_TILE = N_PAD // NS
SPAN_PIECES = [(i * CHUNK, CHUNK) for i in range(12)] + [(6144, 256)]


def _mesh():
    return plsc.VectorSubcoreMesh(core_axis_name="c", subcore_axis_name="s",
                                  num_cores=NC, num_subcores=NS)


_SC_PARAMS = pltpu.CompilerParams(use_tc_tiling_on_sc=False)


def _h_gather_body(elem_hbm, table_hbm, h_hbm, idx_a, idx_b, rows_v, sem):
    c = lax.axis_index("c")
    s = lax.axis_index("s")
    wid = c * NS + s
    base = pl.multiple_of(wid * ROWS_W, 128)
    idx = [idx_a, idx_b]
    pltpu.sync_copy(elem_hbm.at[pl.ds(base, 128)], idx_a)
    descs = [None, None]
    for g in range(NGROUP_W):
        nb = (g + 1) % 2
        if g + 1 < NGROUP_W:
            if descs[nb] is not None:
                descs[nb].wait()
                descs[nb] = None
            pltpu.sync_copy(elem_hbm.at[pl.ds(base + (g + 1) * 128, 128)],
                            idx[nb])
        descs[g % 2] = pltpu.async_copy(
            table_hbm.at[idx[g % 2]],
            rows_v.at[pl.ds(g * 128, 128), :], sem)
    for d in descs:
        if d is not None:
            d.wait()
    pltpu.sync_copy(rows_v, h_hbm.at[pl.ds(base, ROWS_W), :])


def _h_gather(elem_pad, table):
    return pl.kernel(
        _h_gather_body,
        out_type=jax.ShapeDtypeStruct((N_PAD, EMBED_DIM), jnp.float32),
        mesh=_mesh(),
        scratch_types=[
            pltpu.VMEM((128,), jnp.int32),
            pltpu.VMEM((128,), jnp.int32),
            pltpu.VMEM((ROWS_W, EMBED_DIM), jnp.float32),
            pltpu.SemaphoreType.DMA,
        ],
        compiler_params=_SC_PARAMS,
    )(elem_pad, table)


_R_MIN = 0.5 * ANGSTROM_TO_BOHR
_WIDTH = (R_MAX_BOHR - _R_MIN) / NUM_RBF
_CENTERS = np.linspace(_R_MIN, R_MAX_BOHR, NUM_RBF, dtype=np.float32)


def _filter_body(d_ref, w1_ref, b1_ref, w2_ref, b2_ref, out_ref):
    d = d_ref[0, 0, :]
    step = np.float32((R_MAX_BOHR - _R_MIN) / (NUM_RBF - 1))
    k = lax.broadcasted_iota(jnp.int32, (EBLK, NUM_RBF), 1).astype(jnp.float32)
    centers = np.float32(_R_MIN) + k * step
    t = (d[:, None] - centers) * np.float32(1.0 / _WIDTH)
    rbf = jnp.exp(np.float32(-0.5) * t * t)
    z = jnp.dot(rbf, w1_ref[...],
                preferred_element_type=jnp.float32) + b1_ref[0, :]
    zs = z * (np.float32(1.0) / (np.float32(1.0) + jnp.exp(-z)))
    out_ref[...] = jnp.dot(zs, w2_ref[...],
                           preferred_element_type=jnp.float32) + b2_ref[0, :]


def _filter_net(d2, W1, b1, W2, b2):
    nblk = E_PAD // EBLK
    return pl.pallas_call(
        _filter_body,
        grid=(nblk,),
        in_specs=[
            pl.BlockSpec((1, 1, EBLK), lambda i: (i, 0, 0)),
            pl.BlockSpec((NUM_RBF, EMBED_DIM), lambda i: (0, 0)),
            pl.BlockSpec((1, EMBED_DIM), lambda i: (0, 0)),
            pl.BlockSpec((EMBED_DIM, EMBED_DIM), lambda i: (0, 0)),
            pl.BlockSpec((1, EMBED_DIM), lambda i: (0, 0)),
        ],
        out_specs=pl.BlockSpec((EBLK, EMBED_DIM), lambda i: (i, 0)),
        out_shape=jax.ShapeDtypeStruct((E_PAD, EMBED_DIM), jnp.float32),
    )(d2, W1, b1.reshape(1, EMBED_DIM), W2, b2.reshape(1, EMBED_DIM))


def _edge_body(src_hbm, tgt_hbm, wf_hbm, h_hbm, p_hbm,
               idx_t, idx_s, rows, wfv, acc, sem, sem2):
    c = lax.axis_index("c")
    s = lax.axis_index("s")
    wid = c * NS + s

    def zero_row(i, carry):
        rows[i, :] = jnp.zeros((EMBED_DIM,), jnp.float32)
        return carry
    lax.fori_loop(0, CHUNK, zero_row, 0)
    for off, size in SPAN_PIECES:
        pltpu.sync_copy(rows.at[pl.ds(0, size), :],
                        acc.at[pl.ds(s * ROWS_TILE + off, size), :])
    plsc.subcore_barrier()

    base_group = wid * (EDGES_W // 128)

    @pl.loop(0, NCHUNK)
    def chunk_body(ch):
        grow = base_group + ch * GPC
        ebase = grow * 128
        pltpu.sync_copy(tgt_hbm.at[pl.ds(grow, GPC), :], idx_t)
        pltpu.sync_copy(src_hbm.at[pl.ds(grow, GPC), :], idx_s)
        pltpu.sync_copy(wf_hbm.at[pl.ds(ebase, CHUNK), :], wfv)
        descs = []
        for g in range(GPC):
            descs.append(pltpu.async_copy(
                h_hbm.at[idx_t.at[g]],
                rows.at[pl.ds(g * 128, 128), :], sem))
        sdescs = []
        for g in range(GPC):
            descs[g].wait()

            @pl.loop(0, 128, unroll=8)
            def mul_row(e):
                rows[g * 128 + e, :] = rows[g * 128 + e, :] * wfv[g * 128 + e, :]

            sdescs.append(pltpu.async_copy(
                rows.at[pl.ds(g * 128, 128), :],
                acc.at[idx_s.at[g]], sem2, add=True))
        for d in sdescs:
            d.wait()

    plsc.subcore_barrier()

    for off, size in SPAN_PIECES:
        pltpu.sync_copy(acc.at[pl.ds(s * ROWS_TILE + off, size), :],
                        rows.at[pl.ds(0, size), :])
        pltpu.sync_copy(rows.at[pl.ds(0, size), :],
                        p_hbm.at[c, pl.ds(s * ROWS_TILE + off, size), :])


def _edge_aggregate(src2d, tgt2d, wf, h_pad):
    return pl.kernel(
        _edge_body,
        out_type=jax.ShapeDtypeStruct((NC, N_PAD, EMBED_DIM), jnp.float32),
        mesh=_mesh(),
        scratch_types=[
            pltpu.VMEM((GPC, 128), jnp.int32),
            pltpu.VMEM((GPC, 128), jnp.int32),
            pltpu.VMEM((CHUNK, EMBED_DIM), jnp.float32),
            pltpu.VMEM((CHUNK, EMBED_DIM), jnp.float32),
            pltpu.VMEM_SHARED((N_PAD, EMBED_DIM), jnp.float32),
            pltpu.SemaphoreType.DMA,
            pltpu.SemaphoreType.DMA,
        ],
        compiler_params=_SC_PARAMS,
    )(src2d, tgt2d, wf, h_pad)


def _combine_body(h_ref, p_ref, out_ref):
    out_ref[...] = h_ref[...] + p_ref[0] + p_ref[1]


def _combine(h_pad, partials):
    h2 = h_pad.reshape(N_PAD // 8, 128)
    p2 = partials.reshape(NC, N_PAD // 8, 128)
    nrow = N_PAD // 8
    blk = 1280
    out = pl.pallas_call(
        _combine_body,
        grid=(nrow // blk,),
        in_specs=[
            pl.BlockSpec((blk, 128), lambda i: (i, 0)),
            pl.BlockSpec((NC, blk, 128), lambda i: (0, i, 0)),
        ],
        out_specs=pl.BlockSpec((blk, 128), lambda i: (i, 0)),
        out_shape=jax.ShapeDtypeStruct((nrow, 128), jnp.float32),
    )(h2, p2)
    return out.reshape(N_PAD, EMBED_DIM)[:N_NODES]


@jax.jit
def kernel(elem_idx, edge_index, distances, embed_table, W1, b1, W2, b2):
    elem_pad = jnp.pad(elem_idx, (0, N_PAD - N_NODES))

    src = edge_index[0]
    tgt = edge_index[1]
    epad = E_PAD - N_EDGES
    src_pad = jnp.pad(src, (0, epad), constant_values=N_NODES)
    tgt_pad = jnp.pad(tgt, (0, epad))
    d_pad = jnp.pad(distances, (0, epad))
    src2d = src_pad.reshape(E_PAD // 128, 128)
    tgt2d = tgt_pad.reshape(E_PAD // 128, 128)
    d2 = d_pad.reshape(E_PAD // EBLK, 1, EBLK)

    h_pad = _h_gather(elem_pad, embed_table)
    wf = _filter_net(d2, W1, b1, W2, b2)
    partials = _edge_aggregate(src2d, tgt2d, wf, h_pad)
    return _combine(h_pad, partials)
